# baseline (device time: 23804 ns/iter reference)
import jax
import jax.numpy as jnp
import numpy as np
from jax import lax
from jax.experimental import pallas as pl
from jax.experimental.pallas import tpu as pltpu

N_DEV = 4
DH = 64


def kernel(x, Wq, Wo, K_ext, V_ext):
    B, Sq, D = x.shape
    Hq_per = Wq.shape[1] // DH
    Skv = K_ext.shape[1]
    M = B * Sq
    bf16 = jnp.bfloat16
    f32 = jnp.float32

    i = lax.axis_index("i")
    K_loc = lax.dynamic_slice_in_dim(K_ext, i * Hq_per, Hq_per, axis=2)
    V_loc = lax.dynamic_slice_in_dim(V_ext, i * Hq_per, Hq_per, axis=2)
    K_loc = K_loc.transpose(0, 2, 1, 3).astype(bf16)
    V_loc = V_loc.transpose(0, 2, 1, 3).astype(bf16)
    ONES8 = jnp.asarray(np.kron(np.eye(Hq_per), np.ones((Skv, 1))), bf16)
    REP8 = jnp.asarray(np.kron(np.eye(Hq_per), np.ones((1, DH))), bf16)
    Wq_b = (Wq * 0.125).astype(bf16)
    Wo_b = Wo.astype(bf16)

    def body(x_ref, wq_ref, wo_ref, k_ref, v_ref, ones8_ref, rep8_ref,
             out_ref,
             xbuf, acc, sendb, recvb, wkbd, vst,
             ag_send, ag_recv, rs_send, rs_recv):
        my = lax.axis_index("i")
        left = (my - 1) % N_DEV
        right = (my + 1) % N_DEV
        diag = (my + 2) % N_DEV


        barrier = pltpu.get_barrier_semaphore()
        for nbr in (left, right, diag):
            pl.semaphore_signal(barrier, inc=1, device_id=(nbr,),
                                device_id_type=pl.DeviceIdType.MESH)
        xbuf[0, :Sq] = x_ref[0].astype(bf16)
        xbuf[0, Sq:] = x_ref[1].astype(bf16)
        pl.semaphore_wait(barrier, 3)

        def rdma(src, dst, ssem, rsem, tgt):
            return pltpu.make_async_remote_copy(
                src_ref=src, dst_ref=dst, send_sem=ssem, recv_sem=rsem,
                device_id=(tgt,), device_id_type=pl.DeviceIdType.MESH)

        ag = [
            rdma(xbuf.at[0], xbuf.at[1], ag_send.at[0], ag_recv.at[0], right),
            rdma(xbuf.at[0], xbuf.at[3], ag_send.at[1], ag_recv.at[1], left),
            rdma(xbuf.at[0], xbuf.at[2], ag_send.at[2], ag_recv.at[2], diag),
        ]
        for d in ag:
            d.start()

        for r in range(B):
            for h in range(Hq_per):
                blk = lax.dot_general(
                    wq_ref[:, h * DH:(h + 1) * DH], k_ref[r, h],
                    (((1,), (1,)), ((), ())),
                    preferred_element_type=f32)
                wkbd[r, :, h * Skv:(h + 1) * Skv] = blk.astype(bf16)
            vst[r] = jnp.zeros((Hq_per * Skv, Hq_per * DH), bf16)
            for h in range(Hq_per):
                vst[r, h * Skv:(h + 1) * Skv, h * DH:(h + 1) * DH] = (
                    v_ref[r, h])

        def compute(s):
            onorm = [None, None]
            for r in (0, 1):
                xr = xbuf[s, r * Sq:(r + 1) * Sq]
                sc = jnp.dot(xr, wkbd[r],
                             preferred_element_type=f32)
                p = jnp.exp(sc.astype(bf16))
                o = jnp.dot(p, vst[r],
                            preferred_element_type=f32)
                d8 = jnp.dot(p, ones8_ref[...],
                             preferred_element_type=f32)
                den = jnp.dot(d8.astype(bf16), rep8_ref[...],
                              preferred_element_type=f32)
                onorm[r] = (o / den).astype(bf16)
            ostack = jnp.concatenate(onorm, axis=0)
            acc[s] = jnp.dot(ostack, wo_ref[...], preferred_element_type=f32)

        def rs(j, s, tgt):
            sendb[j] = acc[s].astype(bf16)
            return rdma(sendb.at[j], recvb.at[j],
                        rs_send.at[j], rs_recv.at[j], tgt)

        compute(0)
        ag[0].wait()
        compute(1)
        rs0 = rs(0, 1, left)
        rs0.start()
        ag[2].wait()
        compute(2)
        rs2 = rs(2, 2, diag)
        rs2.start()
        ag[1].wait()
        compute(3)
        rs1 = rs(1, 3, right)
        rs1.start()

        rs0.wait()
        rs1.wait()
        rs2.wait()
        final = (acc[0] + recvb[0].astype(f32)
                 + recvb[1].astype(f32) + recvb[2].astype(f32))
        out_ref[0] = final[:Sq]
        out_ref[1] = final[Sq:]

    return pl.pallas_call(
        body,
        out_shape=jax.ShapeDtypeStruct((B, Sq, D), f32),
        in_specs=[pl.BlockSpec(memory_space=pltpu.VMEM)] * 7,
        out_specs=pl.BlockSpec(memory_space=pltpu.VMEM),
        scratch_shapes=[
            pltpu.VMEM((N_DEV, M, D), bf16),
            pltpu.VMEM((N_DEV, M, D), f32),
            pltpu.VMEM((N_DEV - 1, M, D), bf16),
            pltpu.VMEM((N_DEV - 1, M, D), bf16),
            pltpu.VMEM((B, D, Hq_per * Skv), bf16),
            pltpu.VMEM((B, Hq_per * Skv, Hq_per * DH), bf16),
            pltpu.SemaphoreType.DMA((N_DEV - 1,)),
            pltpu.SemaphoreType.DMA((N_DEV - 1,)),
            pltpu.SemaphoreType.DMA((N_DEV - 1,)),
            pltpu.SemaphoreType.DMA((N_DEV - 1,)),
        ],
        compiler_params=pltpu.CompilerParams(collective_id=0),
    )(x, Wq_b, Wo_b, K_loc, V_loc, ONES8, REP8)


# device time: 21874 ns/iter; 1.0882x vs baseline; 1.0882x over previous
import jax
import jax.numpy as jnp
import numpy as np
from jax import lax
from jax.experimental import pallas as pl
from jax.experimental.pallas import tpu as pltpu

N_DEV = 4
DH = 64


def kernel(x, Wq, Wo, K_ext, V_ext):
    B, Sq, D = x.shape
    Hq_per = Wq.shape[1] // DH
    Skv = K_ext.shape[1]
    M = B * Sq
    bf16 = jnp.bfloat16
    f32 = jnp.float32

    i = lax.axis_index("i")
    K_loc = lax.dynamic_slice_in_dim(K_ext, i * Hq_per, Hq_per, axis=2)
    V_loc = lax.dynamic_slice_in_dim(V_ext, i * Hq_per, Hq_per, axis=2)
    K_loc = K_loc.transpose(0, 2, 1, 3).astype(bf16)
    V_loc = V_loc.transpose(0, 2, 1, 3).astype(bf16)
    ONES8 = jnp.asarray(np.kron(np.eye(Hq_per), np.ones((Skv, 1))), bf16)
    REP8 = jnp.asarray(np.kron(np.eye(Hq_per), np.ones((1, DH))), bf16)
    Wq_b = (Wq * 0.125).astype(bf16)
    Wo_b = Wo.astype(bf16)

    def body(x_ref, wq_ref, wo_ref, k_ref, v_ref, ones8_ref, rep8_ref,
             out_ref,
             xbuf, acc, sendb, recvb, wkbd, vst,
             ag_send, ag_recv, rs_send, rs_recv):
        my = lax.axis_index("i")
        left = (my - 1) % N_DEV
        right = (my + 1) % N_DEV
        diag = (my + 2) % N_DEV


        barrier = pltpu.get_barrier_semaphore()
        for nbr in (left, right, diag):
            pl.semaphore_signal(barrier, inc=1, device_id=(nbr,),
                                device_id_type=pl.DeviceIdType.MESH)
        xbuf[0, :Sq] = x_ref[0].astype(bf16)
        xbuf[0, Sq:] = x_ref[1].astype(bf16)
        pl.semaphore_wait(barrier, 3)

        def rdma(src, dst, ssem, rsem, tgt):
            return pltpu.make_async_remote_copy(
                src_ref=src, dst_ref=dst, send_sem=ssem, recv_sem=rsem,
                device_id=(tgt,), device_id_type=pl.DeviceIdType.MESH)

        ag = [
            rdma(xbuf.at[0], xbuf.at[1], ag_send.at[0], ag_recv.at[0], right),
            rdma(xbuf.at[0], xbuf.at[3], ag_send.at[1], ag_recv.at[1], left),
            rdma(xbuf.at[0], xbuf.at[2], ag_send.at[2], ag_recv.at[2], diag),
        ]
        for d in ag:
            d.start()

        for r in range(B):
            for h in range(Hq_per):
                blk = lax.dot_general(
                    wq_ref[:, h * DH:(h + 1) * DH], k_ref[r, h],
                    (((1,), (1,)), ((), ())),
                    preferred_element_type=f32)
                wkbd[r, :, h * Skv:(h + 1) * Skv] = blk.astype(bf16)
            vst[r] = jnp.zeros((Hq_per * Skv, Hq_per * DH), bf16)
            for h in range(Hq_per):
                vst[r, h * Skv:(h + 1) * Skv, h * DH:(h + 1) * DH] = (
                    v_ref[r, h])

        def compute(s):
            onorm = [None, None]
            for r in (0, 1):
                xr = xbuf[s, r * Sq:(r + 1) * Sq]
                sc = jnp.dot(xr, wkbd[r],
                             preferred_element_type=f32)
                p = jnp.exp(sc.astype(bf16))
                o = jnp.dot(p, vst[r],
                            preferred_element_type=f32)
                d8 = jnp.dot(p, ones8_ref[...],
                             preferred_element_type=f32)
                den = jnp.dot(d8.astype(bf16), rep8_ref[...],
                              preferred_element_type=f32)
                onorm[r] = (o / den).astype(bf16)
            ostack = jnp.concatenate(onorm, axis=0)
            acc[s] = jnp.dot(ostack, wo_ref[...], preferred_element_type=f32)

        def rs(j, s, tgt):
            sendb[j] = acc[s].astype(bf16)
            return rdma(sendb.at[j], recvb.at[j],
                        rs_send.at[j], rs_recv.at[j], tgt)

        compute(0)
        ag[0].wait()
        compute(1)
        rs0 = rs(0, 1, left)
        rs0.start()
        ag[1].wait()
        compute(3)
        rs1 = rs(1, 3, right)
        rs1.start()
        ag[2].wait()
        compute(2)
        rs2 = rs(2, 2, diag)
        rs2.start()

        rs0.wait()
        rs1.wait()
        rs2.wait()
        final = (acc[0] + recvb[0].astype(f32)
                 + recvb[1].astype(f32) + recvb[2].astype(f32))
        out_ref[0] = final[:Sq]
        out_ref[1] = final[Sq:]

    return pl.pallas_call(
        body,
        out_shape=jax.ShapeDtypeStruct((B, Sq, D), f32),
        in_specs=[pl.BlockSpec(memory_space=pltpu.VMEM)] * 7,
        out_specs=pl.BlockSpec(memory_space=pltpu.VMEM),
        scratch_shapes=[
            pltpu.VMEM((N_DEV, M, D), bf16),
            pltpu.VMEM((N_DEV, M, D), f32),
            pltpu.VMEM((N_DEV - 1, M, D), bf16),
            pltpu.VMEM((N_DEV - 1, M, D), bf16),
            pltpu.VMEM((B, D, Hq_per * Skv), bf16),
            pltpu.VMEM((B, Hq_per * Skv, Hq_per * DH), bf16),
            pltpu.SemaphoreType.DMA((N_DEV - 1,)),
            pltpu.SemaphoreType.DMA((N_DEV - 1,)),
            pltpu.SemaphoreType.DMA((N_DEV - 1,)),
            pltpu.SemaphoreType.DMA((N_DEV - 1,)),
        ],
        compiler_params=pltpu.CompilerParams(collective_id=0),
    )(x, Wq_b, Wo_b, K_loc, V_loc, ONES8, REP8)


# device time: 21440 ns/iter; 1.1103x vs baseline; 1.0202x over previous
import jax
import jax.numpy as jnp
import numpy as np
from jax import lax
from jax.experimental import pallas as pl
from jax.experimental.pallas import tpu as pltpu

N_DEV = 4
DH = 64


def kernel(x, Wq, Wo, K_ext, V_ext):
    B, Sq, D = x.shape
    Hq_per = Wq.shape[1] // DH
    Skv = K_ext.shape[1]
    M = B * Sq
    bf16 = jnp.bfloat16
    f32 = jnp.float32

    i = lax.axis_index("i")
    K_loc = lax.dynamic_slice_in_dim(K_ext, i * Hq_per, Hq_per, axis=2)
    V_loc = lax.dynamic_slice_in_dim(V_ext, i * Hq_per, Hq_per, axis=2)
    K_loc = K_loc.transpose(0, 2, 1, 3).astype(bf16)
    V_loc = V_loc.transpose(0, 2, 1, 3).astype(bf16)
    ONES8 = jnp.asarray(np.kron(np.eye(Hq_per), np.ones((Skv, 1))), bf16)
    REP8 = jnp.asarray(np.kron(np.eye(Hq_per), np.ones((1, DH))), bf16)
    Wq_b = (Wq * 0.125).astype(bf16)
    Wo_b = Wo.astype(bf16)

    def body(x_ref, wq_ref, wo_ref, k_ref, v_ref, ones8_ref, rep8_ref,
             out_ref,
             xbuf, acc, sendb, recvb, wkbd, vst,
             ag_send, ag_recv, rs_send, rs_recv):
        my = lax.axis_index("i")
        left = (my - 1) % N_DEV
        right = (my + 1) % N_DEV
        diag = (my + 2) % N_DEV


        barrier = pltpu.get_barrier_semaphore()
        for nbr in (left, right, diag):
            pl.semaphore_signal(barrier, inc=1, device_id=(nbr,),
                                device_id_type=pl.DeviceIdType.MESH)
        xbuf[0, :Sq] = x_ref[0].astype(bf16)
        xbuf[0, Sq:] = x_ref[1].astype(bf16)
        pl.semaphore_wait(barrier, 3)

        def rdma(src, dst, ssem, rsem, tgt):
            return pltpu.make_async_remote_copy(
                src_ref=src, dst_ref=dst, send_sem=ssem, recv_sem=rsem,
                device_id=(tgt,), device_id_type=pl.DeviceIdType.MESH)

        ag = [
            rdma(xbuf.at[0], xbuf.at[1], ag_send.at[0], ag_recv.at[0], right),
            rdma(xbuf.at[0], xbuf.at[3], ag_send.at[1], ag_recv.at[1], left),
            rdma(xbuf.at[0], xbuf.at[2], ag_send.at[2], ag_recv.at[2], diag),
        ]
        for d in ag:
            d.start()

        for r in range(B):
            for h in range(Hq_per):
                blk = lax.dot_general(
                    wq_ref[:, h * DH:(h + 1) * DH], k_ref[r, h],
                    (((1,), (1,)), ((), ())),
                    preferred_element_type=f32)
                wkbd[r, :, h * Skv:(h + 1) * Skv] = blk.astype(bf16)
            vst[r] = jnp.zeros((Hq_per * Skv, Hq_per * DH), bf16)
            for h in range(Hq_per):
                vst[r, h * Skv:(h + 1) * Skv, h * DH:(h + 1) * DH] = (
                    v_ref[r, h])

        def compute(s):
            onorm = [None, None]
            for r in (0, 1):
                xr = xbuf[s, r * Sq:(r + 1) * Sq]
                sc = jnp.dot(xr, wkbd[r],
                             preferred_element_type=f32)
                p = jnp.exp(sc.astype(bf16))
                o = jnp.dot(p, vst[r],
                            preferred_element_type=f32)
                d8 = jnp.dot(p, ones8_ref[...],
                             preferred_element_type=f32)
                den = jnp.dot(d8.astype(bf16), rep8_ref[...],
                              preferred_element_type=f32)
                onorm[r] = (o / den).astype(bf16)
            ostack = jnp.concatenate(onorm, axis=0)
            acc[s] = jnp.dot(ostack, wo_ref[...], preferred_element_type=f32)

        def rs(j, s, tgt):
            sendb[j] = acc[s].astype(bf16)
            return rdma(sendb.at[j], recvb.at[j],
                        rs_send.at[j], rs_recv.at[j], tgt)

        compute(0)
        ag[0].wait()
        compute(1)
        rs0 = rs(0, 1, left)
        rs0.start()
        ag[1].wait()
        compute(3)
        rs1 = rs(1, 3, right)
        rs1.start()
        ag[2].wait()
        compute(2)
        rs2 = rs(2, 2, diag)
        rs2.start()

        rs0.wait()
        rs1.wait()
        pre = acc[0] + recvb[0].astype(f32) + recvb[1].astype(f32)
        rs2.wait()
        final = pre + recvb[2].astype(f32)
        out_ref[0] = final[:Sq]
        out_ref[1] = final[Sq:]

    return pl.pallas_call(
        body,
        out_shape=jax.ShapeDtypeStruct((B, Sq, D), f32),
        in_specs=[pl.BlockSpec(memory_space=pltpu.VMEM)] * 7,
        out_specs=pl.BlockSpec(memory_space=pltpu.VMEM),
        scratch_shapes=[
            pltpu.VMEM((N_DEV, M, D), bf16),
            pltpu.VMEM((N_DEV, M, D), f32),
            pltpu.VMEM((N_DEV - 1, M, D), bf16),
            pltpu.VMEM((N_DEV - 1, M, D), bf16),
            pltpu.VMEM((B, D, Hq_per * Skv), bf16),
            pltpu.VMEM((B, Hq_per * Skv, Hq_per * DH), bf16),
            pltpu.SemaphoreType.DMA((N_DEV - 1,)),
            pltpu.SemaphoreType.DMA((N_DEV - 1,)),
            pltpu.SemaphoreType.DMA((N_DEV - 1,)),
            pltpu.SemaphoreType.DMA((N_DEV - 1,)),
        ],
        compiler_params=pltpu.CompilerParams(collective_id=0),
    )(x, Wq_b, Wo_b, K_loc, V_loc, ONES8, REP8)


# device time: 21414 ns/iter; 1.1116x vs baseline; 1.0012x over previous
import jax
import jax.numpy as jnp
import numpy as np
from jax import lax
from jax.experimental import pallas as pl
from jax.experimental.pallas import tpu as pltpu

N_DEV = 4
DH = 64


def kernel(x, Wq, Wo, K_ext, V_ext):
    B, Sq, D = x.shape
    Hq_per = Wq.shape[1] // DH
    Skv = K_ext.shape[1]
    M = B * Sq
    bf16 = jnp.bfloat16
    f32 = jnp.float32

    i = lax.axis_index("i")
    K_loc = lax.dynamic_slice_in_dim(K_ext, i * Hq_per, Hq_per, axis=2)
    V_loc = lax.dynamic_slice_in_dim(V_ext, i * Hq_per, Hq_per, axis=2)
    K_loc = K_loc.transpose(0, 2, 1, 3).astype(bf16)
    V_loc = V_loc.transpose(0, 2, 1, 3).astype(bf16)
    ONES8 = jnp.asarray(np.kron(np.eye(Hq_per), np.ones((Skv, 1))), bf16)
    REP8 = jnp.asarray(np.kron(np.eye(Hq_per), np.ones((1, DH))), bf16)
    Wq_b = (Wq * 0.125).astype(bf16)
    Wo_b = Wo.astype(bf16)

    def body(x_ref, wq_ref, wo_ref, k_ref, v_ref, ones8_ref, rep8_ref,
             out_ref,
             xbuf, acc, sendb, recvb, wkbd, vst,
             ag_send, ag_recv, rs_send, rs_recv):
        my = lax.axis_index("i")
        left = (my - 1) % N_DEV
        right = (my + 1) % N_DEV
        diag = (my + 2) % N_DEV


        barrier = pltpu.get_barrier_semaphore()
        for nbr in (left, right, diag):
            pl.semaphore_signal(barrier, inc=1, device_id=(nbr,),
                                device_id_type=pl.DeviceIdType.MESH)
        xbuf[0, :Sq] = x_ref[0].astype(bf16)
        xbuf[0, Sq:] = x_ref[1].astype(bf16)
        pl.semaphore_wait(barrier, 3)

        def rdma(src, dst, ssem, rsem, tgt):
            return pltpu.make_async_remote_copy(
                src_ref=src, dst_ref=dst, send_sem=ssem, recv_sem=rsem,
                device_id=(tgt,), device_id_type=pl.DeviceIdType.MESH)

        ag = [
            rdma(xbuf.at[0], xbuf.at[1], ag_send.at[0], ag_recv.at[0], right),
            rdma(xbuf.at[0], xbuf.at[3], ag_send.at[1], ag_recv.at[1], left),
            rdma(xbuf.at[0], xbuf.at[2], ag_send.at[2], ag_recv.at[2], diag),
        ]
        for d in ag:
            d.start()

        for r in range(B):
            for h in range(Hq_per):
                blk = lax.dot_general(
                    wq_ref[:, h * DH:(h + 1) * DH], k_ref[r, h],
                    (((1,), (1,)), ((), ())),
                    preferred_element_type=f32)
                wkbd[r, :, h * Skv:(h + 1) * Skv] = blk.astype(bf16)
            vst[r] = jnp.zeros((Hq_per * Skv, Hq_per * DH), bf16)
            for h in range(Hq_per):
                vst[r, h * Skv:(h + 1) * Skv, h * DH:(h + 1) * DH] = (
                    v_ref[r, h])

        def compute(s):
            onorm = [None, None]
            for r in (0, 1):
                xr = xbuf[s, r * Sq:(r + 1) * Sq]
                sc = jnp.dot(xr, wkbd[r],
                             preferred_element_type=f32)
                p = jnp.exp(sc.astype(bf16))
                o = jnp.dot(p, vst[r],
                            preferred_element_type=f32)
                d8 = jnp.dot(p, ones8_ref[...],
                             preferred_element_type=f32)
                den = jnp.dot(d8.astype(bf16), rep8_ref[...],
                              preferred_element_type=f32)
                onorm[r] = (o / den).astype(bf16)
            ostack = jnp.concatenate(onorm, axis=0)
            acc[s] = jnp.dot(ostack, wo_ref[...], preferred_element_type=f32)

        def compute_stream(s, j, tgt):
            descs = []
            for r in (0, 1):
                rows = slice(r * Sq, (r + 1) * Sq)
                xr = xbuf[s, rows]
                sc = jnp.dot(xr, wkbd[r],
                             preferred_element_type=f32)
                p = jnp.exp(sc.astype(bf16))
                o = jnp.dot(p, vst[r],
                            preferred_element_type=f32)
                d8 = jnp.dot(p, ones8_ref[...],
                             preferred_element_type=f32)
                den = jnp.dot(d8.astype(bf16), rep8_ref[...],
                              preferred_element_type=f32)
                onorm = (o / den).astype(bf16)
                part = jnp.dot(onorm, wo_ref[...],
                               preferred_element_type=f32)
                sendb[j, rows] = part.astype(bf16)
                d = rdma(sendb.at[j, rows], recvb.at[j, rows],
                         rs_send.at[j, r], rs_recv.at[j, r], tgt)
                d.start()
                descs.append(d)
            return descs

        compute(0)
        ag[0].wait()
        rs0 = compute_stream(1, 0, left)
        ag[1].wait()
        rs1 = compute_stream(3, 1, right)
        ag[2].wait()
        rs2 = compute_stream(2, 2, diag)

        for d in rs0 + rs1:
            d.wait()
        pre = acc[0] + recvb[0].astype(f32) + recvb[1].astype(f32)
        for d in rs2:
            d.wait()
        final = pre + recvb[2].astype(f32)
        out_ref[0] = final[:Sq]
        out_ref[1] = final[Sq:]

    return pl.pallas_call(
        body,
        out_shape=jax.ShapeDtypeStruct((B, Sq, D), f32),
        in_specs=[pl.BlockSpec(memory_space=pltpu.VMEM)] * 7,
        out_specs=pl.BlockSpec(memory_space=pltpu.VMEM),
        scratch_shapes=[
            pltpu.VMEM((N_DEV, M, D), bf16),
            pltpu.VMEM((N_DEV, M, D), f32),
            pltpu.VMEM((N_DEV - 1, M, D), bf16),
            pltpu.VMEM((N_DEV - 1, M, D), bf16),
            pltpu.VMEM((B, D, Hq_per * Skv), bf16),
            pltpu.VMEM((B, Hq_per * Skv, Hq_per * DH), bf16),
            pltpu.SemaphoreType.DMA((N_DEV - 1,)),
            pltpu.SemaphoreType.DMA((N_DEV - 1,)),
            pltpu.SemaphoreType.DMA((N_DEV - 1, 2)),
            pltpu.SemaphoreType.DMA((N_DEV - 1, 2)),
        ],
        compiler_params=pltpu.CompilerParams(collective_id=0),
    )(x, Wq_b, Wo_b, K_loc, V_loc, ONES8, REP8)
